# speedup vs baseline: 4.7530x; 4.7530x over previous
import functools
import math

import jax
import jax.numpy as jnp
from jax import lax
from jax.experimental import pallas as pl
from jax.experimental.pallas import tpu as pltpu
from jax.experimental.pallas import tpu_sc as plsc

VOCAB = 100000
D = 128
SCALE = math.sqrt(128.0)

_info = plsc.get_sparse_core_info()
_NC, _NS = _info.num_cores, _info.num_subcores
NW = _NC * _NS

B = 4096 * 200
B_PER_W = B // NW
CHUNK = 128
N_CHUNKS = B_PER_W // CHUNK


_SCALE_ROWS = 2000


def _scale_body(t_ref, o_ref):
    o_ref[...] = t_ref[...] * SCALE


_scale_table = pl.pallas_call(
    _scale_body,
    out_shape=jax.ShapeDtypeStruct((VOCAB, D), jnp.float32),
    grid=(VOCAB // _SCALE_ROWS,),
    in_specs=[pl.BlockSpec((_SCALE_ROWS, D), lambda i: (i, 0))],
    out_specs=pl.BlockSpec((_SCALE_ROWS, D), lambda i: (i, 0)),
)


@functools.partial(
    pl.kernel,
    mesh=plsc.VectorSubcoreMesh(core_axis_name="c", subcore_axis_name="s"),
    out_type=jax.ShapeDtypeStruct((B, D), jnp.float32),
    scratch_types=[
        pltpu.VMEM((CHUNK,), jnp.int32),
        pltpu.VMEM((CHUNK, D), jnp.float32),
        pltpu.SemaphoreType.DMA,
    ],
)
def _gather(idx_hbm, table_hbm, out_hbm, idx_v, rows_v, sem):
    wid = lax.axis_index("s") * _NC + lax.axis_index("c")
    base = wid * B_PER_W

    def body(i, carry):
        off = base + i * CHUNK
        pltpu.sync_copy(idx_hbm.at[pl.ds(off, CHUNK)], idx_v)
        pltpu.async_copy(table_hbm.at[idx_v], rows_v, sem).wait()
        pltpu.sync_copy(rows_v, out_hbm.at[pl.ds(off, CHUNK)])
        return carry

    lax.fori_loop(0, N_CHUNKS, body, 0)


def kernel(x, table):
    xf = x.reshape(-1).astype(jnp.int32)
    scaled = _scale_table(table)
    out = _gather(xf, scaled)
    return out.reshape(x.shape[0], x.shape[1], D)


# speedup vs baseline: 7.9484x; 1.6723x over previous
import functools
import math

import jax
import jax.numpy as jnp
from jax import lax
from jax.experimental import pallas as pl
from jax.experimental.pallas import tpu as pltpu
from jax.experimental.pallas import tpu_sc as plsc

VOCAB = 100000
D = 128
SCALE = math.sqrt(128.0)

_info = plsc.get_sparse_core_info()
_NC, _NS = _info.num_cores, _info.num_subcores
NW = _NC * _NS

B = 4096 * 200
B_PER_W = B // NW
CHUNK = 128
N_CHUNKS = B_PER_W // CHUNK
NBUF = 4
LOOKAHEAD = 2


_SCALE_ROWS = 2000


def _scale_body(t_ref, o_ref):
    o_ref[...] = t_ref[...] * SCALE


_scale_table = pl.pallas_call(
    _scale_body,
    out_shape=jax.ShapeDtypeStruct((VOCAB, D), jnp.float32),
    grid=(VOCAB // _SCALE_ROWS,),
    in_specs=[pl.BlockSpec((_SCALE_ROWS, D), lambda i: (i, 0))],
    out_specs=pl.BlockSpec((_SCALE_ROWS, D), lambda i: (i, 0)),
)


@functools.partial(
    pl.kernel,
    mesh=plsc.VectorSubcoreMesh(core_axis_name="c", subcore_axis_name="s"),
    out_type=jax.ShapeDtypeStruct((B, D), jnp.float32),
    scratch_types=(
        [pltpu.VMEM((N_CHUNKS, CHUNK), jnp.int32),
         pltpu.VMEM((NBUF, CHUNK, D), jnp.float32)]
        + [pltpu.SemaphoreType.DMA] * (2 * NBUF)
    ),
)
def _gather(idx_hbm, table_hbm, out_hbm, idx_v, rows_v, *sems):
    gsem = sems[:NBUF]
    osem = sems[NBUF:]
    wid = lax.axis_index("s") * _NC + lax.axis_index("c")
    base = wid * B_PER_W

    pltpu.sync_copy(idx_hbm.at[wid], idx_v)

    def start_gather(c, b):
        pltpu.async_copy(table_hbm.at[idx_v.at[c]], rows_v.at[b], gsem[b])

    def wait_gather(c, b):
        pltpu.make_async_copy(
            table_hbm.at[idx_v.at[c]], rows_v.at[b], gsem[b]).wait()

    def start_out(c, b):
        pltpu.async_copy(
            rows_v.at[b], out_hbm.at[pl.ds(base + c * CHUNK, CHUNK)], osem[b])

    def wait_out(c, b):
        pltpu.make_async_copy(
            rows_v.at[b], out_hbm.at[pl.ds(base + c * CHUNK, CHUNK)],
            osem[b]).wait()

    for c in range(LOOKAHEAD):
        start_gather(c, c % NBUF)

    def outer(g, carry):
        for b in range(NBUF):
            c = g * NBUF + b
            wait_gather(c, b)
            start_out(c, b)
            c2 = c + LOOKAHEAD
            b2 = (b + LOOKAHEAD) % NBUF

            @pl.when(jnp.logical_and(c2 >= NBUF, c2 < N_CHUNKS))
            def _():
                wait_out(c2 - NBUF, b2)

            @pl.when(c2 < N_CHUNKS)
            def _():
                start_gather(c2, b2)
        return carry

    lax.fori_loop(0, N_CHUNKS // NBUF, outer, 0)

    for c in range(N_CHUNKS - LOOKAHEAD, N_CHUNKS):
        wait_out(c, c % NBUF)


def kernel(x, table):
    xf = x.reshape(NW, N_CHUNKS, CHUNK).astype(jnp.int32)
    scaled = _scale_table(table)
    out = _gather(xf, scaled)
    return out.reshape(x.shape[0], x.shape[1], D)


# speedup vs baseline: 7.9628x; 1.0018x over previous
import functools
import math

import jax
import jax.numpy as jnp
from jax import lax
from jax.experimental import pallas as pl
from jax.experimental.pallas import tpu as pltpu
from jax.experimental.pallas import tpu_sc as plsc

VOCAB = 100000
D = 128
SCALE = math.sqrt(128.0)

_info = plsc.get_sparse_core_info()
_NC, _NS = _info.num_cores, _info.num_subcores
NW = _NC * _NS

B = 4096 * 200
B_PER_W = B // NW
CHUNK = 128
N_CHUNKS = B_PER_W // CHUNK
NBUF = 5
LOOKAHEAD = 3


_SCALE_ROWS = 2000


def _scale_body(t_ref, o_ref):
    o_ref[...] = t_ref[...] * SCALE


_scale_table = pl.pallas_call(
    _scale_body,
    out_shape=jax.ShapeDtypeStruct((VOCAB, D), jnp.float32),
    grid=(VOCAB // _SCALE_ROWS,),
    in_specs=[pl.BlockSpec((_SCALE_ROWS, D), lambda i: (i, 0))],
    out_specs=pl.BlockSpec((_SCALE_ROWS, D), lambda i: (i, 0)),
)


@functools.partial(
    pl.kernel,
    mesh=plsc.VectorSubcoreMesh(core_axis_name="c", subcore_axis_name="s"),
    out_type=jax.ShapeDtypeStruct((B, D), jnp.float32),
    scratch_types=(
        [pltpu.VMEM((N_CHUNKS, CHUNK), jnp.int32),
         pltpu.VMEM((NBUF, CHUNK, D), jnp.float32)]
        + [pltpu.SemaphoreType.DMA] * (2 * NBUF)
    ),
)
def _gather(idx_hbm, table_hbm, out_hbm, idx_v, rows_v, *sems):
    gsem = sems[:NBUF]
    osem = sems[NBUF:]
    wid = lax.axis_index("s") * _NC + lax.axis_index("c")
    base = wid * B_PER_W

    pltpu.sync_copy(idx_hbm.at[wid], idx_v)

    def start_gather(c, b):
        pltpu.async_copy(table_hbm.at[idx_v.at[c]], rows_v.at[b], gsem[b])

    def wait_gather(c, b):
        pltpu.make_async_copy(
            table_hbm.at[idx_v.at[c]], rows_v.at[b], gsem[b]).wait()

    def start_out(c, b):
        pltpu.async_copy(
            rows_v.at[b], out_hbm.at[pl.ds(base + c * CHUNK, CHUNK)], osem[b])

    def wait_out(c, b):
        pltpu.make_async_copy(
            rows_v.at[b], out_hbm.at[pl.ds(base + c * CHUNK, CHUNK)],
            osem[b]).wait()

    for c in range(LOOKAHEAD):
        start_gather(c, c % NBUF)

    def outer(g, carry):
        for b in range(NBUF):
            c = g * NBUF + b
            wait_gather(c, b)
            start_out(c, b)
            c2 = c + LOOKAHEAD
            b2 = (b + LOOKAHEAD) % NBUF

            @pl.when(jnp.logical_and(c2 >= NBUF, c2 < N_CHUNKS))
            def _():
                wait_out(c2 - NBUF, b2)

            @pl.when(c2 < N_CHUNKS)
            def _():
                start_gather(c2, b2)
        return carry

    lax.fori_loop(0, N_CHUNKS // NBUF, outer, 0)

    for c in range(N_CHUNKS - LOOKAHEAD, N_CHUNKS):
        wait_out(c, c % NBUF)


def kernel(x, table):
    xf = x.reshape(NW, N_CHUNKS, CHUNK).astype(jnp.int32)
    scaled = _scale_table(table)
    out = _gather(xf, scaled)
    return out.reshape(x.shape[0], x.shape[1], D)


# speedup vs baseline: 9.1821x; 1.1531x over previous
import functools
import math

import jax
import jax.numpy as jnp
from jax import lax
from jax.experimental import pallas as pl
from jax.experimental.pallas import tpu as pltpu
from jax.experimental.pallas import tpu_sc as plsc

VOCAB = 100000
D = 128
SCALE = math.sqrt(128.0)

_info = plsc.get_sparse_core_info()
_NC, _NS = _info.num_cores, _info.num_subcores
NW = _NC * _NS
_L = 16

B = 4096 * 200
B_PER_W = B // NW
CHUNK = 128
N_CHUNKS = B_PER_W // CHUNK
NBUF = 5
LOOKAHEAD = 3


@functools.partial(
    pl.kernel,
    mesh=plsc.VectorSubcoreMesh(core_axis_name="c", subcore_axis_name="s"),
    out_type=jax.ShapeDtypeStruct((B, D), jnp.float32),
    scratch_types=(
        [pltpu.VMEM((N_CHUNKS, CHUNK), jnp.int32),
         pltpu.VMEM((NBUF, CHUNK, D), jnp.float32)]
        + [pltpu.SemaphoreType.DMA] * (2 * NBUF)
    ),
)
def _gather(idx_hbm, table_hbm, out_hbm, idx_v, rows_v, *sems):
    gsem = sems[:NBUF]
    osem = sems[NBUF:]
    wid = lax.axis_index("s") * _NC + lax.axis_index("c")
    base = wid * B_PER_W

    pltpu.sync_copy(idx_hbm.at[wid], idx_v)

    def start_gather(c, b):
        pltpu.async_copy(table_hbm.at[idx_v.at[c]], rows_v.at[b], gsem[b])

    def wait_gather(c, b):
        pltpu.make_async_copy(
            table_hbm.at[idx_v.at[c]], rows_v.at[b], gsem[b]).wait()

    def start_out(c, b):
        pltpu.async_copy(
            rows_v.at[b], out_hbm.at[pl.ds(base + c * CHUNK, CHUNK)], osem[b])

    def wait_out(c, b):
        pltpu.make_async_copy(
            rows_v.at[b], out_hbm.at[pl.ds(base + c * CHUNK, CHUNK)],
            osem[b]).wait()

    def scale_buf(b):
        @plsc.parallel_loop(0, CHUNK, step=1, unroll=2)
        def _(r):
            for k in range(D // _L):
                rows_v[b, r, pl.ds(k * _L, _L)] = (
                    rows_v[b, r, pl.ds(k * _L, _L)] * SCALE)

    for c in range(LOOKAHEAD):
        start_gather(c, c % NBUF)

    def outer(g, carry):
        for b in range(NBUF):
            c = g * NBUF + b
            wait_gather(c, b)
            scale_buf(b)
            start_out(c, b)
            c2 = c + LOOKAHEAD
            b2 = (b + LOOKAHEAD) % NBUF

            @pl.when(jnp.logical_and(c2 >= NBUF, c2 < N_CHUNKS))
            def _():
                wait_out(c2 - NBUF, b2)

            @pl.when(c2 < N_CHUNKS)
            def _():
                start_gather(c2, b2)
        return carry

    lax.fori_loop(0, N_CHUNKS // NBUF, outer, 0)

    for c in range(N_CHUNKS - LOOKAHEAD, N_CHUNKS):
        wait_out(c, c % NBUF)


def kernel(x, table):
    xf = x.reshape(NW, N_CHUNKS, CHUNK).astype(jnp.int32)
    out = _gather(xf, table)
    return out.reshape(x.shape[0], x.shape[1], D)
